# baseline (speedup 1.0000x reference)
import functools

import jax
import jax.numpy as jnp
from jax.experimental import pallas as pl
from jax.experimental.pallas import tpu as pltpu

N_HASHES = 4
CHUNK = 144
RES_SCALE = 1.0
REDUCTION = 4
RANK_TILE = 512


def _conv(x, w, b):
    out = jax.lax.conv_general_dilated(x, w, (1, 1), 'SAME',
                                       dimension_numbers=('NCHW', 'OIHW', 'NCHW'))
    return out + b[None, :, None, None]


def _rank_kernel(codes_ref, tri_ref, r1_ref, hist_ref, carry_ref, *, nb):
    t = pl.program_id(1)

    @pl.when(t == 0)
    def _():
        carry_ref[...] = jnp.zeros_like(carry_ref)

    c = codes_ref[0, :, :]
    lanes = jax.lax.broadcasted_iota(jnp.int32, (1, nb), 1)
    onehot = (c == lanes).astype(jnp.float32)

    cum = jax.lax.dot_general(tri_ref[...], onehot.astype(jnp.bfloat16),
                              (((1,), (0,)), ((), ())),
                              preferred_element_type=jnp.float32)
    carry = carry_ref[0:1, :]
    r1 = jnp.sum(onehot * (cum + carry), axis=1, keepdims=True)
    r1_ref[0, :, :] = r1.astype(jnp.int32)

    new_carry = carry + jnp.sum(onehot, axis=0, keepdims=True)
    carry_ref[0:1, :] = new_carry
    hist_ref[0, :, :] = new_carry


def _pos_kernel(codes_ref, r1_ref, base_ref, pos_ref, *, nb, L):
    h = pl.program_id(0)
    c = codes_ref[0, :, :]
    lanes = jax.lax.broadcasted_iota(jnp.int32, (1, nb), 1)
    onehot = (c == lanes).astype(jnp.float32)
    base = base_ref[0, :, :].astype(jnp.float32)
    b = jnp.sum(onehot * base, axis=1, keepdims=True)
    pos_ref[0, :, :] = (b.astype(jnp.int32) + r1_ref[0, :, :] + h * L)


def _counting_rank(codes, L):
    H, Lpad = codes.shape
    T = RANK_TILE
    nt = Lpad // T
    nb = 128
    codes3 = codes.reshape(H * nt, T, 1)
    ii = jax.lax.broadcasted_iota(jnp.int32, (T, T), 0)
    jj = jax.lax.broadcasted_iota(jnp.int32, (T, T), 1)
    tri = (jj < ii).astype(jnp.bfloat16)

    r1, hist = pl.pallas_call(
        functools.partial(_rank_kernel, nb=nb),
        grid=(H, nt),
        in_specs=[
            pl.BlockSpec((1, T, 1), lambda h, t: (h * nt + t, 0, 0)),
            pl.BlockSpec((T, T), lambda h, t: (0, 0)),
        ],
        out_specs=[
            pl.BlockSpec((1, T, 1), lambda h, t: (h * nt + t, 0, 0)),
            pl.BlockSpec((1, 1, nb), lambda h, t: (h * nt + t, 0, 0)),
        ],
        out_shape=[
            jax.ShapeDtypeStruct((H * nt, T, 1), jnp.int32),
            jax.ShapeDtypeStruct((H * nt, 1, nb), jnp.float32),
        ],
        scratch_shapes=[pltpu.VMEM((8, nb), jnp.float32)],
    )(codes3, tri)

    totals = hist.reshape(H, nt, nb)[:, -1, :]
    bucket_base = (jnp.cumsum(totals, axis=1) - totals).astype(jnp.int32)
    base3 = bucket_base.reshape(H, 1, nb)

    pos = pl.pallas_call(
        functools.partial(_pos_kernel, nb=nb, L=L),
        grid=(H, nt),
        in_specs=[
            pl.BlockSpec((1, T, 1), lambda h, t: (h * nt + t, 0, 0)),
            pl.BlockSpec((1, T, 1), lambda h, t: (h * nt + t, 0, 0)),
            pl.BlockSpec((1, 1, nb), lambda h, t: (h, 0, 0)),
        ],
        out_specs=pl.BlockSpec((1, T, 1), lambda h, t: (h * nt + t, 0, 0)),
        out_shape=jax.ShapeDtypeStruct((H * nt, T, 1), jnp.int32),
    )(codes3, r1, base3)
    return pos.reshape(H, Lpad)


def _attn_kernel(xys_ref, out_ref, lse_ref, *, nc, C, Cr):
    k = pl.program_id(1)

    prev = jax.lax.rem(k - 1 + nc, nc)
    nxt = jax.lax.rem(k + 1, nc)

    def load(j):
        return xys_ref[0, pl.ds(j * CHUNK, CHUNK), :]

    t_c = load(k)
    t_p = load(prev)
    t_n = load(nxt)
    q = t_c[:, :C]

    def score_block(t):
        kx = t[:, :C]
        nrm = jnp.sqrt(jnp.sum(kx * kx, axis=1, keepdims=True))
        kn = kx / jnp.maximum(nrm, 5e-05)
        return jax.lax.dot_general(q, kn, (((1,), (1,)), ((), ())),
                                   preferred_element_type=jnp.float32)

    s_c = score_block(t_c)
    s_p = score_block(t_p)
    s_n = score_block(t_n)

    m = jnp.maximum(jnp.maximum(jnp.max(s_c, axis=1), jnp.max(s_p, axis=1)),
                    jnp.max(s_n, axis=1))[:, None]
    p_c = jnp.exp(s_c - m)
    p_p = jnp.exp(s_p - m)
    p_n = jnp.exp(s_n - m)
    denom = (jnp.sum(p_c, axis=1) + jnp.sum(p_p, axis=1)
             + jnp.sum(p_n, axis=1))[:, None]

    def pv(p, t):
        return jax.lax.dot_general(p, t[:, C:], (((1,), (0,)), ((), ())),
                                   preferred_element_type=jnp.float32)

    acc = pv(p_c, t_c) + pv(p_p, t_p) + pv(p_n, t_n)
    out_ref[0, :, :] = acc / denom
    lse_ref[0, :, :] = m + jnp.log(denom)


def _banded_attention(xys, C, Cr):
    H, Lp, _ = xys.shape
    nc = Lp // CHUNK
    kern = functools.partial(_attn_kernel, nc=nc, C=C, Cr=Cr)
    out, lse = pl.pallas_call(
        kern,
        grid=(H, nc),
        in_specs=[
            pl.BlockSpec((1, Lp, C + Cr), lambda h, k: (h, 0, 0)),
        ],
        out_specs=[
            pl.BlockSpec((1, CHUNK, Cr), lambda h, k: (h, k, 0)),
            pl.BlockSpec((1, CHUNK, 1), lambda h, k: (h, k, 0)),
        ],
        out_shape=[
            jax.ShapeDtypeStruct((H, Lp, Cr), jnp.float32),
            jax.ShapeDtypeStruct((H, Lp, 1), jnp.float32),
        ],
    )(xys)
    return out, lse[..., 0]


def kernel(input, w_match, b_match, w_assembly, b_assembly):
    x = input
    N, _, Hh, Ww = x.shape
    L = Hh * Ww
    C = w_match.shape[0]
    Cr = w_assembly.shape[0]
    H = N_HASHES

    x_embed = _conv(x, w_match, b_match).reshape(N, C, L).transpose(0, 2, 1)
    y_embed = _conv(x, w_assembly, b_assembly).reshape(N, Cr, L).transpose(0, 2, 1)

    hash_buckets = min(L // CHUNK + (L // CHUNK) % 2, 128)
    rot = jax.random.normal(jax.random.key(42), (1, C, H, hash_buckets // 2),
                            dtype=x_embed.dtype)
    rot = jnp.broadcast_to(rot, (N, C, H, hash_buckets // 2))
    rotated = jnp.einsum('btf,bfhi->bhti', x_embed, rot)
    rotated = jnp.concatenate([rotated, -rotated], axis=-1)
    codes = jnp.argmax(rotated, axis=-1).astype(jnp.int32)

    codes2 = codes.reshape(H, L)
    rank_pad = (-L) % RANK_TILE
    if rank_pad:
        codes2 = jnp.concatenate(
            [codes2, jnp.full((H, rank_pad), 128, jnp.int32)], axis=1)
    pos = _counting_rank(codes2, L)[:, :L]
    undo_sort = pos.reshape(N, H * L)

    flat_ids = jnp.broadcast_to(
        jnp.arange(L, dtype=jnp.int32)[None, :], (H, L)).reshape(-1)
    mod_indices = jnp.zeros((H * L,), jnp.int32).at[pos.reshape(-1)].set(
        flat_ids, unique_indices=True).reshape(N, H * L)

    xy_embed = jnp.concatenate([x_embed, y_embed], axis=-1)
    xy_sorted = jnp.take_along_axis(xy_embed, mod_indices[:, :, None], axis=1)

    padding = CHUNK - L % CHUNK if L % CHUNK != 0 else 0
    xys = xy_sorted.reshape(N * H, L, C + Cr)
    if padding:
        xys = jnp.concatenate([xys, xys[:, -padding:, :]], axis=1)

    ret, bucket_score = _banded_attention(xys, C, Cr)

    if padding:
        ret = ret[:, :-padding, :]
        bucket_score = bucket_score[:, :-padding]
    ret = ret.reshape(N, H * L, Cr)
    bucket_score = bucket_score.reshape(N, H * L)

    ret = jnp.take_along_axis(ret, undo_sort[:, :, None], axis=1)
    bucket_score = jnp.take_along_axis(bucket_score, undo_sort, axis=1)

    ret = ret.reshape(N, H, L, Cr)
    bucket_score = bucket_score.reshape(N, H, L, 1)
    probs = jax.nn.softmax(bucket_score, axis=1)
    ret = jnp.sum(ret * probs, axis=1)
    ret = ret.transpose(0, 2, 1).reshape(N, Cr, Hh, Ww) * RES_SCALE + x
    return ret


# speedup vs baseline: 1.1813x; 1.1813x over previous
import functools

import jax
import jax.numpy as jnp
from jax.experimental import pallas as pl
from jax.experimental.pallas import tpu as pltpu

N_HASHES = 4
CHUNK = 144
RES_SCALE = 1.0
REDUCTION = 4
RANK_TILE = 1024
ATTN_BLOCK = 8


def _conv(x, w, b):
    out = jax.lax.conv_general_dilated(x, w, (1, 1), 'SAME',
                                       dimension_numbers=('NCHW', 'OIHW', 'NCHW'))
    return out + b[None, :, None, None]


def _rank_kernel(codes_ref, tri_ref, r1_ref, hist_ref, carry_ref, *, nb, H):
    t = pl.program_id(0)

    @pl.when(t == 0)
    def _():
        carry_ref[...] = jnp.zeros_like(carry_ref)

    c = codes_ref[0, :, :]
    lanes = jax.lax.broadcasted_iota(jnp.int32, (1, nb), 1)
    oh = jnp.concatenate(
        [(c[:, h:h + 1] == lanes) for h in range(H)],
        axis=1).astype(jnp.float32)

    cum = jax.lax.dot_general(tri_ref[...], oh.astype(jnp.bfloat16),
                              (((1,), (0,)), ((), ())),
                              preferred_element_type=jnp.float32)
    carry = carry_ref[0:1, :]
    tot = oh * (cum + carry)
    r1 = jnp.concatenate(
        [jnp.sum(tot[:, h * nb:(h + 1) * nb], axis=1, keepdims=True)
         for h in range(H)], axis=1)
    r1_ref[0, :, :] = r1.astype(jnp.int32)

    new_carry = carry + jnp.sum(oh, axis=0, keepdims=True)
    carry_ref[0:1, :] = new_carry
    hist_ref[0, :, :] = new_carry


def _pos_kernel(codes_ref, r1_ref, base_ref, pos_ref, *, nb, H):
    c = codes_ref[0, :, :]
    lanes = jax.lax.broadcasted_iota(jnp.int32, (1, nb), 1)
    oh = jnp.concatenate(
        [(c[:, h:h + 1] == lanes) for h in range(H)],
        axis=1).astype(jnp.float32)
    base = base_ref[0, :, :].astype(jnp.float32)
    bl = oh * base
    b = jnp.concatenate(
        [jnp.sum(bl[:, h * nb:(h + 1) * nb], axis=1, keepdims=True)
         for h in range(H)], axis=1)
    pos_ref[0, :, :] = b.astype(jnp.int32) + r1_ref[0, :, :]


def _counting_rank(codes4, L, H):
    nt, T, _ = codes4.shape
    nb = 128
    ii = jax.lax.broadcasted_iota(jnp.int32, (T, T), 0)
    jj = jax.lax.broadcasted_iota(jnp.int32, (T, T), 1)
    tri = (jj < ii).astype(jnp.bfloat16)

    r1, hist = pl.pallas_call(
        functools.partial(_rank_kernel, nb=nb, H=H),
        grid=(nt,),
        in_specs=[
            pl.BlockSpec((1, T, H), lambda t: (t, 0, 0)),
            pl.BlockSpec((T, T), lambda t: (0, 0)),
        ],
        out_specs=[
            pl.BlockSpec((1, T, H), lambda t: (t, 0, 0)),
            pl.BlockSpec((1, 1, H * nb), lambda t: (0, 0, 0)),
        ],
        out_shape=[
            jax.ShapeDtypeStruct((nt, T, H), jnp.int32),
            jax.ShapeDtypeStruct((1, 1, H * nb), jnp.float32),
        ],
        scratch_shapes=[pltpu.VMEM((8, H * nb), jnp.float32)],
    )(codes4, tri)

    totals = hist.reshape(H, nb)
    bucket_base = (jnp.cumsum(totals, axis=1) - totals
                   + (jnp.arange(H, dtype=jnp.float32) * L)[:, None])
    base3 = bucket_base.astype(jnp.int32).reshape(1, 1, H * nb)

    pos = pl.pallas_call(
        functools.partial(_pos_kernel, nb=nb, H=H),
        grid=(nt,),
        in_specs=[
            pl.BlockSpec((1, T, H), lambda t: (t, 0, 0)),
            pl.BlockSpec((1, T, H), lambda t: (t, 0, 0)),
            pl.BlockSpec((1, 1, H * nb), lambda t: (0, 0, 0)),
        ],
        out_specs=pl.BlockSpec((1, T, H), lambda t: (t, 0, 0)),
        out_shape=jax.ShapeDtypeStruct((nt, T, H), jnp.int32),
    )(codes4, r1, base3)
    return pos


def _attn_kernel(xys_ref, out_ref, lse_ref, *, nc, C, Cr, G):
    g = pl.program_id(1)

    def load(j):
        return xys_ref[0, pl.ds(j * CHUNK, CHUNK), :]

    for c in range(G):
        k = g * G + c

        @pl.when(k < nc)
        def _(k=k, c=c):
            prev = jax.lax.rem(k - 1 + nc, nc)
            nxt = jax.lax.rem(k + 1, nc)
            t_c = load(k)
            t_p = load(prev)
            t_n = load(nxt)
            q = t_c[:, :C]

            def score_block(t):
                kx = t[:, :C]
                nrm = jnp.sqrt(jnp.sum(kx * kx, axis=1, keepdims=True))
                kn = kx / jnp.maximum(nrm, 5e-05)
                return jax.lax.dot_general(q, kn, (((1,), (1,)), ((), ())),
                                           preferred_element_type=jnp.float32)

            s_c = score_block(t_c)
            s_p = score_block(t_p)
            s_n = score_block(t_n)

            m = jnp.maximum(jnp.maximum(jnp.max(s_c, axis=1),
                                        jnp.max(s_p, axis=1)),
                            jnp.max(s_n, axis=1))[:, None]
            p_c = jnp.exp(s_c - m)
            p_p = jnp.exp(s_p - m)
            p_n = jnp.exp(s_n - m)
            denom = (jnp.sum(p_c, axis=1) + jnp.sum(p_p, axis=1)
                     + jnp.sum(p_n, axis=1))[:, None]

            def pv(p, t):
                return jax.lax.dot_general(p, t[:, C:], (((1,), (0,)), ((), ())),
                                           preferred_element_type=jnp.float32)

            acc = pv(p_c, t_c) + pv(p_p, t_p) + pv(p_n, t_n)
            out_ref[0, pl.ds(c * CHUNK, CHUNK), :] = acc / denom
            lse_ref[0, pl.ds(c * CHUNK, CHUNK), :] = m + jnp.log(denom)


def _banded_attention(xys, C, Cr):
    H, Lp, _ = xys.shape
    nc = Lp // CHUNK
    G = ATTN_BLOCK
    gb = -(-nc // G)
    Lo = gb * G * CHUNK
    kern = functools.partial(_attn_kernel, nc=nc, C=C, Cr=Cr, G=G)
    out, lse = pl.pallas_call(
        kern,
        grid=(H, gb),
        in_specs=[
            pl.BlockSpec((1, Lp, C + Cr), lambda h, g: (h, 0, 0)),
        ],
        out_specs=[
            pl.BlockSpec((1, G * CHUNK, Cr), lambda h, g: (h, g, 0)),
            pl.BlockSpec((1, G * CHUNK, 1), lambda h, g: (h, g, 0)),
        ],
        out_shape=[
            jax.ShapeDtypeStruct((H, Lo, Cr), jnp.float32),
            jax.ShapeDtypeStruct((H, Lo, 1), jnp.float32),
        ],
    )(xys)
    return out[:, :Lp], lse[:, :Lp, 0]


def kernel(input, w_match, b_match, w_assembly, b_assembly):
    x = input
    N, _, Hh, Ww = x.shape
    L = Hh * Ww
    C = w_match.shape[0]
    Cr = w_assembly.shape[0]
    H = N_HASHES

    x_embed = _conv(x, w_match, b_match).reshape(N, C, L).transpose(0, 2, 1)
    y_embed = _conv(x, w_assembly, b_assembly).reshape(N, Cr, L).transpose(0, 2, 1)

    hash_buckets = min(L // CHUNK + (L // CHUNK) % 2, 128)
    rot = jax.random.normal(jax.random.key(42), (1, C, H, hash_buckets // 2),
                            dtype=x_embed.dtype)
    rot = jnp.broadcast_to(rot, (N, C, H, hash_buckets // 2))
    rotated = jnp.einsum('btf,bfhi->bhti', x_embed, rot)
    rotated = jnp.concatenate([rotated, -rotated], axis=-1)
    codes = jnp.argmax(rotated, axis=-1).astype(jnp.int32)

    T = RANK_TILE
    rank_pad = (-L) % T
    codes_lh = codes.reshape(H, L).transpose(1, 0)
    if rank_pad:
        codes_lh = jnp.concatenate(
            [codes_lh, jnp.full((rank_pad, H), 128, jnp.int32)], axis=0)
    nt = codes_lh.shape[0] // T
    pos4 = _counting_rank(codes_lh.reshape(nt, T, H), L, H)
    pos = pos4.reshape(nt * T, H)[:L].transpose(1, 0)
    undo_sort = pos.reshape(N, H * L)

    flat_ids = jnp.broadcast_to(
        jnp.arange(L, dtype=jnp.int32)[None, :], (H, L)).reshape(-1)
    mod_indices = jnp.zeros((H * L,), jnp.int32).at[pos.reshape(-1)].set(
        flat_ids, unique_indices=True).reshape(N, H * L)

    xy_embed = jnp.concatenate([x_embed, y_embed], axis=-1)
    xy_sorted = jnp.take_along_axis(xy_embed, mod_indices[:, :, None], axis=1)

    padding = CHUNK - L % CHUNK if L % CHUNK != 0 else 0
    xys = xy_sorted.reshape(N * H, L, C + Cr)
    if padding:
        xys = jnp.concatenate([xys, xys[:, -padding:, :]], axis=1)

    ret, bucket_score = _banded_attention(xys, C, Cr)

    if padding:
        ret = ret[:, :-padding, :]
        bucket_score = bucket_score[:, :-padding]
    ret = ret.reshape(N, H * L, Cr)
    bucket_score = bucket_score.reshape(N, H * L)

    ret = jnp.take_along_axis(ret, undo_sort[:, :, None], axis=1)
    bucket_score = jnp.take_along_axis(bucket_score, undo_sort, axis=1)

    ret = ret.reshape(N, H, L, Cr)
    bucket_score = bucket_score.reshape(N, H, L, 1)
    probs = jax.nn.softmax(bucket_score, axis=1)
    ret = jnp.sum(ret * probs, axis=1)
    ret = ret.transpose(0, 2, 1).reshape(N, Cr, Hh, Ww) * RES_SCALE + x
    return ret


# speedup vs baseline: 2.0130x; 1.7040x over previous
import functools

import jax
import jax.numpy as jnp
from jax.experimental import pallas as pl
from jax.experimental.pallas import tpu as pltpu
from jax.experimental.pallas import tpu_sc as plsc

N_HASHES = 4
CHUNK = 144
RES_SCALE = 1.0
REDUCTION = 4
RANK_TILE = 1024
ATTN_BLOCK = 8


def _conv(x, w, b):
    out = jax.lax.conv_general_dilated(x, w, (1, 1), 'SAME',
                                       dimension_numbers=('NCHW', 'OIHW', 'NCHW'))
    return out + b[None, :, None, None]


def _rank_kernel(codes_ref, tri_ref, r1_ref, hist_ref, carry_ref, *, nb, H):
    t = pl.program_id(0)

    @pl.when(t == 0)
    def _():
        carry_ref[...] = jnp.zeros_like(carry_ref)

    c = codes_ref[0, :, :]
    lanes = jax.lax.broadcasted_iota(jnp.int32, (1, nb), 1)
    oh = jnp.concatenate(
        [(c[:, h:h + 1] == lanes) for h in range(H)],
        axis=1).astype(jnp.float32)

    cum = jax.lax.dot_general(tri_ref[...], oh.astype(jnp.bfloat16),
                              (((1,), (0,)), ((), ())),
                              preferred_element_type=jnp.float32)
    carry = carry_ref[0:1, :]
    tot = oh * (cum + carry)
    r1 = jnp.concatenate(
        [jnp.sum(tot[:, h * nb:(h + 1) * nb], axis=1, keepdims=True)
         for h in range(H)], axis=1)
    r1_ref[0, :, :] = r1.astype(jnp.int32)

    new_carry = carry + jnp.sum(oh, axis=0, keepdims=True)
    carry_ref[0:1, :] = new_carry
    hist_ref[0, :, :] = new_carry


def _pos_kernel(codes_ref, r1_ref, base_ref, pos_ref, *, nb, H):
    c = codes_ref[0, :, :]
    lanes = jax.lax.broadcasted_iota(jnp.int32, (1, nb), 1)
    oh = jnp.concatenate(
        [(c[:, h:h + 1] == lanes) for h in range(H)],
        axis=1).astype(jnp.float32)
    base = base_ref[0, :, :].astype(jnp.float32)
    bl = oh * base
    b = jnp.concatenate(
        [jnp.sum(bl[:, h * nb:(h + 1) * nb], axis=1, keepdims=True)
         for h in range(H)], axis=1)
    pos_ref[0, :, :] = b.astype(jnp.int32) + r1_ref[0, :, :]


def _counting_rank(codes4, L, H):
    nt, T, _ = codes4.shape
    nb = 128
    ii = jax.lax.broadcasted_iota(jnp.int32, (T, T), 0)
    jj = jax.lax.broadcasted_iota(jnp.int32, (T, T), 1)
    tri = (jj < ii).astype(jnp.bfloat16)

    r1, hist = pl.pallas_call(
        functools.partial(_rank_kernel, nb=nb, H=H),
        grid=(nt,),
        in_specs=[
            pl.BlockSpec((1, T, H), lambda t: (t, 0, 0)),
            pl.BlockSpec((T, T), lambda t: (0, 0)),
        ],
        out_specs=[
            pl.BlockSpec((1, T, H), lambda t: (t, 0, 0)),
            pl.BlockSpec((1, 1, H * nb), lambda t: (0, 0, 0)),
        ],
        out_shape=[
            jax.ShapeDtypeStruct((nt, T, H), jnp.int32),
            jax.ShapeDtypeStruct((1, 1, H * nb), jnp.float32),
        ],
        scratch_shapes=[pltpu.VMEM((8, H * nb), jnp.float32)],
    )(codes4, tri)

    totals = hist.reshape(H, nb)
    bucket_base = (jnp.cumsum(totals, axis=1) - totals
                   + (jnp.arange(H, dtype=jnp.float32) * L)[:, None])
    base3 = bucket_base.astype(jnp.int32).reshape(1, 1, H * nb)

    pos = pl.pallas_call(
        functools.partial(_pos_kernel, nb=nb, H=H),
        grid=(nt,),
        in_specs=[
            pl.BlockSpec((1, T, H), lambda t: (t, 0, 0)),
            pl.BlockSpec((1, T, H), lambda t: (t, 0, 0)),
            pl.BlockSpec((1, 1, H * nb), lambda t: (0, 0, 0)),
        ],
        out_specs=pl.BlockSpec((1, T, H), lambda t: (t, 0, 0)),
        out_shape=jax.ShapeDtypeStruct((nt, T, H), jnp.int32),
    )(codes4, r1, base3)
    return pos


SC_WINDOW = 128


def _sc_sort_scatter(rows, spos, H, Lp):
    L, D = rows.shape
    n_idx = H * L
    nxb = L // SC_WINDOW

    @functools.partial(
        pl.kernel,
        out_type=jax.ShapeDtypeStruct((H * Lp, D), rows.dtype),
        mesh=plsc.VectorSubcoreMesh(core_axis_name="core",
                                    subcore_axis_name="subcore"),
        scratch_types=[])
    def scatter_kernel(x_hbm, i_hbm, o_hbm):
        def body(x_vmem, i_vmem):
            pltpu.sync_copy(x_vmem, o_hbm.at[i_vmem.at[0]])

        pltpu.emit_pipeline(
            body,
            grid=(n_idx // SC_WINDOW,),
            in_specs=[
                pl.BlockSpec((SC_WINDOW, D),
                             index_map=lambda i: (jax.lax.rem(i, nxb), 0)),
                pl.BlockSpec((1, SC_WINDOW), index_map=lambda i: (0, i)),
            ],
            out_specs=[],
            core_axis_name=('core', 'subcore'),
            dimension_semantics=(pltpu.PARALLEL,),
        )(x_hbm, i_hbm)

    return scatter_kernel(rows, spos)


def _attn_kernel(xys_ref, out_ref, lse_ref, *, nc, C, Cr, G):
    g = pl.program_id(1)

    def load(j):
        return xys_ref[0, pl.ds(j * CHUNK, CHUNK), :]

    for c in range(G):
        k = g * G + c

        @pl.when(k < nc)
        def _(k=k, c=c):
            prev = jax.lax.rem(k - 1 + nc, nc)
            nxt = jax.lax.rem(k + 1, nc)
            t_c = load(k)
            t_p = load(prev)
            t_n = load(nxt)
            q = t_c[:, :C]

            def score_block(t):
                kx = t[:, :C]
                nrm = jnp.sqrt(jnp.sum(kx * kx, axis=1, keepdims=True))
                kn = kx / jnp.maximum(nrm, 5e-05)
                return jax.lax.dot_general(q, kn, (((1,), (1,)), ((), ())),
                                           preferred_element_type=jnp.float32)

            s_c = score_block(t_c)
            s_p = score_block(t_p)
            s_n = score_block(t_n)

            m = jnp.maximum(jnp.maximum(jnp.max(s_c, axis=1),
                                        jnp.max(s_p, axis=1)),
                            jnp.max(s_n, axis=1))[:, None]
            p_c = jnp.exp(s_c - m)
            p_p = jnp.exp(s_p - m)
            p_n = jnp.exp(s_n - m)
            denom = (jnp.sum(p_c, axis=1) + jnp.sum(p_p, axis=1)
                     + jnp.sum(p_n, axis=1))[:, None]

            def pv(p, t):
                return jax.lax.dot_general(p, t[:, C:C + Cr],
                                           (((1,), (0,)), ((), ())),
                                           preferred_element_type=jnp.float32)

            acc = pv(p_c, t_c) + pv(p_p, t_p) + pv(p_n, t_n)
            out_ref[0, pl.ds(c * CHUNK, CHUNK), :] = acc / denom
            lse_ref[0, pl.ds(c * CHUNK, CHUNK), :] = m + jnp.log(denom)


def _banded_attention(xys, C, Cr):
    H, Lp, D = xys.shape
    nc = Lp // CHUNK
    G = ATTN_BLOCK
    gb = -(-nc // G)
    Lo = gb * G * CHUNK
    kern = functools.partial(_attn_kernel, nc=nc, C=C, Cr=Cr, G=G)
    out, lse = pl.pallas_call(
        kern,
        grid=(H, gb),
        in_specs=[
            pl.BlockSpec((1, Lp, D), lambda h, g: (h, 0, 0)),
        ],
        out_specs=[
            pl.BlockSpec((1, G * CHUNK, Cr), lambda h, g: (h, g, 0)),
            pl.BlockSpec((1, G * CHUNK, 1), lambda h, g: (h, g, 0)),
        ],
        out_shape=[
            jax.ShapeDtypeStruct((H, Lo, Cr), jnp.float32),
            jax.ShapeDtypeStruct((H, Lo, 1), jnp.float32),
        ],
    )(xys)
    return out[:, :Lp], lse[:, :Lp, 0]


def kernel(input, w_match, b_match, w_assembly, b_assembly):
    x = input
    N, _, Hh, Ww = x.shape
    L = Hh * Ww
    C = w_match.shape[0]
    Cr = w_assembly.shape[0]
    H = N_HASHES

    x_embed = _conv(x, w_match, b_match).reshape(N, C, L).transpose(0, 2, 1)
    y_embed = _conv(x, w_assembly, b_assembly).reshape(N, Cr, L).transpose(0, 2, 1)

    hash_buckets = min(L // CHUNK + (L // CHUNK) % 2, 128)
    rot = jax.random.normal(jax.random.key(42), (1, C, H, hash_buckets // 2),
                            dtype=x_embed.dtype)
    rot = jnp.broadcast_to(rot, (N, C, H, hash_buckets // 2))
    rotated = jnp.einsum('btf,bfhi->bhti', x_embed, rot)
    rotated = jnp.concatenate([rotated, -rotated], axis=-1)
    codes = jnp.argmax(rotated, axis=-1).astype(jnp.int32)

    T = RANK_TILE
    rank_pad = (-L) % T
    codes_lh = codes.reshape(H, L).transpose(1, 0)
    if rank_pad:
        codes_lh = jnp.concatenate(
            [codes_lh, jnp.full((rank_pad, H), 128, jnp.int32)], axis=0)
    nt = codes_lh.shape[0] // T
    pos4 = _counting_rank(codes_lh.reshape(nt, T, H), L, H)
    pos = pos4.reshape(nt * T, H)[:L].transpose(1, 0)
    undo_sort = pos.reshape(N, H * L)

    padding = CHUNK - L % CHUNK if L % CHUNK != 0 else 0
    Lp = L + padding

    spos = (pos + (jnp.arange(H, dtype=jnp.int32) * padding)[:, None]
            ).reshape(1, H * L)

    d_pad = (-(C + Cr)) % 128
    xy_embed = jnp.concatenate(
        [x_embed, y_embed]
        + ([jnp.zeros((N, L, d_pad), x_embed.dtype)] if d_pad else []),
        axis=-1)
    D = C + Cr + d_pad
    xys_flat = _sc_sort_scatter(xy_embed.reshape(L, D), spos, H, Lp)
    xys = xys_flat.reshape(H, Lp, D)
    if padding:
        xys = xys.at[:, L:, :].set(xys[:, L - padding:L, :])

    ret, bucket_score = _banded_attention(xys, C, Cr)

    if padding:
        ret = ret[:, :-padding, :]
        bucket_score = bucket_score[:, :-padding]
    ret = ret.reshape(N, H * L, Cr)
    bucket_score = bucket_score.reshape(N, H * L)

    ret = jnp.take_along_axis(ret, undo_sort[:, :, None], axis=1)
    bucket_score = jnp.take_along_axis(bucket_score, undo_sort, axis=1)

    ret = ret.reshape(N, H, L, Cr)
    bucket_score = bucket_score.reshape(N, H, L, 1)
    probs = jax.nn.softmax(bucket_score, axis=1)
    ret = jnp.sum(ret * probs, axis=1)
    ret = ret.transpose(0, 2, 1).reshape(N, Cr, Hh, Ww) * RES_SCALE + x
    return ret


# speedup vs baseline: 2.0176x; 1.0023x over previous
import functools

import jax
import jax.numpy as jnp
from jax.experimental import pallas as pl
from jax.experimental.pallas import tpu as pltpu
from jax.experimental.pallas import tpu_sc as plsc

N_HASHES = 4
CHUNK = 144
RES_SCALE = 1.0
REDUCTION = 4
RANK_TILE = 1024
ATTN_BLOCK = 8


def _conv(x, w, b):
    out = jax.lax.conv_general_dilated(x, w, (1, 1), 'SAME',
                                       dimension_numbers=('NCHW', 'OIHW', 'NCHW'))
    return out + b[None, :, None, None]


def _rank_kernel(codes_ref, tri_ref, r1_ref, hist_ref, carry_ref, *, nb, H):
    t = pl.program_id(0)

    @pl.when(t == 0)
    def _():
        carry_ref[...] = jnp.zeros_like(carry_ref)

    c = codes_ref[0, :, :]
    lanes = jax.lax.broadcasted_iota(jnp.int32, (1, nb), 1)
    oh = jnp.concatenate(
        [(c[:, h:h + 1] == lanes) for h in range(H)],
        axis=1).astype(jnp.float32)

    cum = jax.lax.dot_general(tri_ref[...], oh.astype(jnp.bfloat16),
                              (((1,), (0,)), ((), ())),
                              preferred_element_type=jnp.float32)
    carry = carry_ref[0:1, :]
    tot = oh * (cum + carry)
    r1 = jnp.concatenate(
        [jnp.sum(tot[:, h * nb:(h + 1) * nb], axis=1, keepdims=True)
         for h in range(H)], axis=1)
    r1_ref[0, :, :] = r1.astype(jnp.int32)

    new_carry = carry + jnp.sum(oh, axis=0, keepdims=True)
    carry_ref[0:1, :] = new_carry
    hist_ref[0, :, :] = new_carry


def _pos_kernel(codes_ref, r1_ref, base_ref, pos_ref, *, nb, H):
    c = codes_ref[0, :, :]
    lanes = jax.lax.broadcasted_iota(jnp.int32, (1, nb), 1)
    oh = jnp.concatenate(
        [(c[:, h:h + 1] == lanes) for h in range(H)],
        axis=1).astype(jnp.float32)
    base = base_ref[0, :, :].astype(jnp.float32)
    bl = oh * base
    b = jnp.concatenate(
        [jnp.sum(bl[:, h * nb:(h + 1) * nb], axis=1, keepdims=True)
         for h in range(H)], axis=1)
    pos_ref[0, :, :] = b.astype(jnp.int32) + r1_ref[0, :, :]


def _counting_rank(codes4, L, H):
    nt, T, _ = codes4.shape
    nb = 128
    ii = jax.lax.broadcasted_iota(jnp.int32, (T, T), 0)
    jj = jax.lax.broadcasted_iota(jnp.int32, (T, T), 1)
    tri = (jj < ii).astype(jnp.bfloat16)

    r1, hist = pl.pallas_call(
        functools.partial(_rank_kernel, nb=nb, H=H),
        grid=(nt,),
        in_specs=[
            pl.BlockSpec((1, T, H), lambda t: (t, 0, 0)),
            pl.BlockSpec((T, T), lambda t: (0, 0)),
        ],
        out_specs=[
            pl.BlockSpec((1, T, H), lambda t: (t, 0, 0)),
            pl.BlockSpec((1, 1, H * nb), lambda t: (0, 0, 0)),
        ],
        out_shape=[
            jax.ShapeDtypeStruct((nt, T, H), jnp.int32),
            jax.ShapeDtypeStruct((1, 1, H * nb), jnp.float32),
        ],
        scratch_shapes=[pltpu.VMEM((8, H * nb), jnp.float32)],
    )(codes4, tri)

    totals = hist.reshape(H, nb)
    bucket_base = (jnp.cumsum(totals, axis=1) - totals
                   + (jnp.arange(H, dtype=jnp.float32) * L)[:, None])
    base3 = bucket_base.astype(jnp.int32).reshape(1, 1, H * nb)

    pos = pl.pallas_call(
        functools.partial(_pos_kernel, nb=nb, H=H),
        grid=(nt,),
        in_specs=[
            pl.BlockSpec((1, T, H), lambda t: (t, 0, 0)),
            pl.BlockSpec((1, T, H), lambda t: (t, 0, 0)),
            pl.BlockSpec((1, 1, H * nb), lambda t: (0, 0, 0)),
        ],
        out_specs=pl.BlockSpec((1, T, H), lambda t: (t, 0, 0)),
        out_shape=jax.ShapeDtypeStruct((nt, T, H), jnp.int32),
    )(codes4, r1, base3)
    return pos


SC_WINDOW = 128


def _sc_sort_scatter(rows, spos, H):
    L, D = rows.shape
    n_idx = H * L
    nxb = L // SC_WINDOW

    @functools.partial(
        pl.kernel,
        out_type=jax.ShapeDtypeStruct((H * L, D), rows.dtype),
        mesh=plsc.VectorSubcoreMesh(core_axis_name="core",
                                    subcore_axis_name="subcore"),
        scratch_types=[])
    def scatter_kernel(x_hbm, i_hbm, o_hbm):
        def body(x_vmem, i_vmem):
            pltpu.sync_copy(x_vmem, o_hbm.at[i_vmem.at[0]])

        pltpu.emit_pipeline(
            body,
            grid=(n_idx // SC_WINDOW,),
            in_specs=[
                pl.BlockSpec((SC_WINDOW, D),
                             index_map=lambda i: (jax.lax.rem(i, nxb), 0)),
                pl.BlockSpec((1, SC_WINDOW), index_map=lambda i: (0, i)),
            ],
            out_specs=[],
            core_axis_name=('core', 'subcore'),
            dimension_semantics=(pltpu.PARALLEL,),
        )(x_hbm, i_hbm)

    return scatter_kernel(rows, spos)


def _attn_kernel(xys_ref, out_ref, lse_ref, *, nc, C, Cr, G, L, padding):
    g = pl.program_id(1)

    def load(j):
        if padding:
            s1 = CHUNK - padding
            a = jnp.where(j == nc - 1, L - s1, j * CHUNK)
            b = jnp.where(j == nc - 1, L - padding, j * CHUNK + s1)
            return jnp.concatenate(
                [xys_ref[0, pl.ds(a, s1), :],
                 xys_ref[0, pl.ds(b, padding), :]], axis=0)
        return xys_ref[0, pl.ds(j * CHUNK, CHUNK), :]

    for c in range(G):
        k = g * G + c

        @pl.when(k < nc)
        def _(k=k, c=c):
            prev = jax.lax.rem(k - 1 + nc, nc)
            nxt = jax.lax.rem(k + 1, nc)
            t_c = load(k)
            t_p = load(prev)
            t_n = load(nxt)
            q = t_c[:, :C]

            def score_block(t):
                kx = t[:, :C]
                nrm = jnp.sqrt(jnp.sum(kx * kx, axis=1, keepdims=True))
                kn = kx / jnp.maximum(nrm, 5e-05)
                return jax.lax.dot_general(q, kn, (((1,), (1,)), ((), ())),
                                           preferred_element_type=jnp.float32)

            s_c = score_block(t_c)
            s_p = score_block(t_p)
            s_n = score_block(t_n)

            m = jnp.maximum(jnp.maximum(jnp.max(s_c, axis=1),
                                        jnp.max(s_p, axis=1)),
                            jnp.max(s_n, axis=1))[:, None]
            p_c = jnp.exp(s_c - m)
            p_p = jnp.exp(s_p - m)
            p_n = jnp.exp(s_n - m)
            denom = (jnp.sum(p_c, axis=1) + jnp.sum(p_p, axis=1)
                     + jnp.sum(p_n, axis=1))[:, None]

            def pv(p, t):
                return jax.lax.dot_general(p, t[:, C:C + Cr],
                                           (((1,), (0,)), ((), ())),
                                           preferred_element_type=jnp.float32)

            acc = pv(p_c, t_c) + pv(p_p, t_p) + pv(p_n, t_n)
            out_ref[0, pl.ds(c * CHUNK, CHUNK), :] = acc / denom
            lse_ref[0, pl.ds(c * CHUNK, CHUNK), :] = m + jnp.log(denom)


def _banded_attention(xys, C, Cr, padding):
    H, L, D = xys.shape
    Lp = L + padding
    nc = Lp // CHUNK
    G = ATTN_BLOCK
    gb = -(-nc // G)
    Lo = gb * G * CHUNK
    kern = functools.partial(_attn_kernel, nc=nc, C=C, Cr=Cr, G=G, L=L,
                             padding=padding)
    out, lse = pl.pallas_call(
        kern,
        grid=(H, gb),
        in_specs=[
            pl.BlockSpec((1, L, D), lambda h, g: (h, 0, 0)),
        ],
        out_specs=[
            pl.BlockSpec((1, G * CHUNK, Cr), lambda h, g: (h, g, 0)),
            pl.BlockSpec((1, G * CHUNK, 1), lambda h, g: (h, g, 0)),
        ],
        out_shape=[
            jax.ShapeDtypeStruct((H, Lo, Cr), jnp.float32),
            jax.ShapeDtypeStruct((H, Lo, 1), jnp.float32),
        ],
    )(xys)
    return out[:, :Lp], lse[:, :Lp, 0]


def kernel(input, w_match, b_match, w_assembly, b_assembly):
    x = input
    N, _, Hh, Ww = x.shape
    L = Hh * Ww
    C = w_match.shape[0]
    Cr = w_assembly.shape[0]
    H = N_HASHES

    x_embed = _conv(x, w_match, b_match).reshape(N, C, L).transpose(0, 2, 1)
    y_embed = _conv(x, w_assembly, b_assembly).reshape(N, Cr, L).transpose(0, 2, 1)

    hash_buckets = min(L // CHUNK + (L // CHUNK) % 2, 128)
    rot = jax.random.normal(jax.random.key(42), (1, C, H, hash_buckets // 2),
                            dtype=x_embed.dtype)
    rot = jnp.broadcast_to(rot, (N, C, H, hash_buckets // 2))
    rotated = jnp.einsum('btf,bfhi->bhti', x_embed, rot)
    rotated = jnp.concatenate([rotated, -rotated], axis=-1)
    codes = jnp.argmax(rotated, axis=-1).astype(jnp.int32)

    T = RANK_TILE
    rank_pad = (-L) % T
    codes_lh = codes.reshape(H, L).transpose(1, 0)
    if rank_pad:
        codes_lh = jnp.concatenate(
            [codes_lh, jnp.full((rank_pad, H), 128, jnp.int32)], axis=0)
    nt = codes_lh.shape[0] // T
    pos4 = _counting_rank(codes_lh.reshape(nt, T, H), L, H)
    pos = pos4.reshape(nt * T, H)[:L].transpose(1, 0)
    undo_sort = pos.reshape(N, H * L)

    padding = CHUNK - L % CHUNK if L % CHUNK != 0 else 0
    Lp = L + padding

    spos = pos.reshape(1, H * L)

    d_pad = (-(C + Cr)) % 128
    xy_embed = jnp.concatenate(
        [x_embed, y_embed]
        + ([jnp.zeros((N, L, d_pad), x_embed.dtype)] if d_pad else []),
        axis=-1)
    D = C + Cr + d_pad
    xys_flat = _sc_sort_scatter(xy_embed.reshape(L, D), spos, H)
    xys = xys_flat.reshape(H, L, D)

    ret, bucket_score = _banded_attention(xys, C, Cr, padding)

    if padding:
        ret = ret[:, :-padding, :]
        bucket_score = bucket_score[:, :-padding]
    ret = ret.reshape(N, H * L, Cr)
    bucket_score = bucket_score.reshape(N, H * L)

    ret = jnp.take_along_axis(ret, undo_sort[:, :, None], axis=1)
    bucket_score = jnp.take_along_axis(bucket_score, undo_sort, axis=1)

    ret = ret.reshape(N, H, L, Cr)
    bucket_score = bucket_score.reshape(N, H, L, 1)
    probs = jax.nn.softmax(bucket_score, axis=1)
    ret = jnp.sum(ret * probs, axis=1)
    ret = ret.transpose(0, 2, 1).reshape(N, Cr, Hh, Ww) * RES_SCALE + x
    return ret


# speedup vs baseline: 2.0191x; 1.0007x over previous
import functools

import jax
import jax.numpy as jnp
from jax.experimental import pallas as pl
from jax.experimental.pallas import tpu as pltpu
from jax.experimental.pallas import tpu_sc as plsc

N_HASHES = 4
CHUNK = 144
RES_SCALE = 1.0
REDUCTION = 4
RANK_TILE = 1024
ATTN_BLOCK = 8


def _conv(x, w, b):
    out = jax.lax.conv_general_dilated(x, w, (1, 1), 'SAME',
                                       dimension_numbers=('NCHW', 'OIHW', 'NCHW'))
    return out + b[None, :, None, None]


def _rank_kernel(codes_ref, tri_ref, r1_ref, hist_ref, carry_ref, *, nb, H):
    t = pl.program_id(0)

    @pl.when(t == 0)
    def _():
        carry_ref[...] = jnp.zeros_like(carry_ref)

    c = codes_ref[0, :, :]
    lanes = jax.lax.broadcasted_iota(jnp.int32, (1, nb), 1)
    oh = jnp.concatenate(
        [(c[:, h:h + 1] == lanes) for h in range(H)],
        axis=1).astype(jnp.float32)

    cum = jax.lax.dot_general(tri_ref[...], oh.astype(jnp.bfloat16),
                              (((1,), (0,)), ((), ())),
                              preferred_element_type=jnp.float32)
    carry = carry_ref[0:1, :]
    tot = oh * (cum + carry)
    r1 = jnp.concatenate(
        [jnp.sum(tot[:, h * nb:(h + 1) * nb], axis=1, keepdims=True)
         for h in range(H)], axis=1)
    r1_ref[0, :, :] = r1.astype(jnp.int32)

    new_carry = carry + jnp.sum(oh, axis=0, keepdims=True)
    carry_ref[0:1, :] = new_carry
    hist_ref[0, :, :] = new_carry


def _pos_kernel(codes_ref, r1_ref, base_ref, pos_ref, *, nb, H):
    c = codes_ref[0, :, :]
    lanes = jax.lax.broadcasted_iota(jnp.int32, (1, nb), 1)
    oh = jnp.concatenate(
        [(c[:, h:h + 1] == lanes) for h in range(H)],
        axis=1).astype(jnp.float32)
    base = base_ref[0, :, :].astype(jnp.float32)
    bl = oh * base
    b = jnp.concatenate(
        [jnp.sum(bl[:, h * nb:(h + 1) * nb], axis=1, keepdims=True)
         for h in range(H)], axis=1)
    pos_ref[0, :, :] = b.astype(jnp.int32) + r1_ref[0, :, :]


def _counting_rank(codes4, L, H):
    nt, T, _ = codes4.shape
    nb = 128
    ii = jax.lax.broadcasted_iota(jnp.int32, (T, T), 0)
    jj = jax.lax.broadcasted_iota(jnp.int32, (T, T), 1)
    tri = (jj < ii).astype(jnp.bfloat16)

    r1, hist = pl.pallas_call(
        functools.partial(_rank_kernel, nb=nb, H=H),
        grid=(nt,),
        in_specs=[
            pl.BlockSpec((1, T, H), lambda t: (t, 0, 0)),
            pl.BlockSpec((T, T), lambda t: (0, 0)),
        ],
        out_specs=[
            pl.BlockSpec((1, T, H), lambda t: (t, 0, 0)),
            pl.BlockSpec((1, 1, H * nb), lambda t: (0, 0, 0)),
        ],
        out_shape=[
            jax.ShapeDtypeStruct((nt, T, H), jnp.int32),
            jax.ShapeDtypeStruct((1, 1, H * nb), jnp.float32),
        ],
        scratch_shapes=[pltpu.VMEM((8, H * nb), jnp.float32)],
    )(codes4, tri)

    totals = hist.reshape(H, nb)
    bucket_base = (jnp.cumsum(totals, axis=1) - totals
                   + (jnp.arange(H, dtype=jnp.float32) * L)[:, None])
    base3 = bucket_base.astype(jnp.int32).reshape(1, 1, H * nb)

    pos = pl.pallas_call(
        functools.partial(_pos_kernel, nb=nb, H=H),
        grid=(nt,),
        in_specs=[
            pl.BlockSpec((1, T, H), lambda t: (t, 0, 0)),
            pl.BlockSpec((1, T, H), lambda t: (t, 0, 0)),
            pl.BlockSpec((1, 1, H * nb), lambda t: (0, 0, 0)),
        ],
        out_specs=pl.BlockSpec((1, T, H), lambda t: (t, 0, 0)),
        out_shape=jax.ShapeDtypeStruct((nt, T, H), jnp.int32),
    )(codes4, r1, base3)
    return pos


SC_WINDOW = 128


def _sc_sort_scatter(rows, spos, H):
    L, D = rows.shape
    n_idx = H * L
    nxb = L // SC_WINDOW

    @functools.partial(
        pl.kernel,
        out_type=jax.ShapeDtypeStruct((H * L, D), rows.dtype),
        mesh=plsc.VectorSubcoreMesh(core_axis_name="core",
                                    subcore_axis_name="subcore"),
        scratch_types=[])
    def scatter_kernel(x_hbm, i_hbm, o_hbm):
        def body(x_vmem, i_vmem):
            pltpu.sync_copy(x_vmem, o_hbm.at[i_vmem.at[0]])

        pltpu.emit_pipeline(
            body,
            grid=(n_idx // SC_WINDOW,),
            in_specs=[
                pl.BlockSpec((SC_WINDOW, D),
                             index_map=lambda i: (jax.lax.rem(i, nxb), 0)),
                pl.BlockSpec((1, SC_WINDOW), index_map=lambda i: (0, i)),
            ],
            out_specs=[],
            core_axis_name=('core', 'subcore'),
            dimension_semantics=(pltpu.PARALLEL,),
        )(x_hbm, i_hbm)

    return scatter_kernel(rows, spos)


def _attn_kernel(xys_ref, out_ref, lse_ref, *, nc, C, Cr, G, L, padding):
    g = pl.program_id(1)

    def load(j):
        if padding:
            s1 = CHUNK - padding
            a = jnp.where(j == nc - 1, L - s1, j * CHUNK)
            b = jnp.where(j == nc - 1, L - padding, j * CHUNK + s1)
            return jnp.concatenate(
                [xys_ref[0, pl.ds(a, s1), :],
                 xys_ref[0, pl.ds(b, padding), :]], axis=0)
        return xys_ref[0, pl.ds(j * CHUNK, CHUNK), :]

    for c in range(G):
        k = g * G + c

        @pl.when(k < nc)
        def _(k=k, c=c):
            prev = jax.lax.rem(k - 1 + nc, nc)
            nxt = jax.lax.rem(k + 1, nc)
            t_c = load(k)
            t_p = load(prev)
            t_n = load(nxt)
            q = t_c[:, :C]

            def score_block(t):
                kx = t[:, :C]
                nrm = jnp.sqrt(jnp.sum(kx * kx, axis=1, keepdims=True))
                kn = kx / jnp.maximum(nrm, 5e-05)
                return jax.lax.dot_general(q, kn, (((1,), (1,)), ((), ())),
                                           preferred_element_type=jnp.float32)

            s_c = score_block(t_c)
            s_p = score_block(t_p)
            s_n = score_block(t_n)

            m = jnp.maximum(jnp.maximum(jnp.max(s_c, axis=1),
                                        jnp.max(s_p, axis=1)),
                            jnp.max(s_n, axis=1))[:, None]
            p_c = jnp.exp(s_c - m)
            p_p = jnp.exp(s_p - m)
            p_n = jnp.exp(s_n - m)
            denom = (jnp.sum(p_c, axis=1) + jnp.sum(p_p, axis=1)
                     + jnp.sum(p_n, axis=1))[:, None]

            def pv(p, t):
                return jax.lax.dot_general(p, t[:, C:C + Cr],
                                           (((1,), (0,)), ((), ())),
                                           preferred_element_type=jnp.float32)

            acc = pv(p_c, t_c) + pv(p_p, t_p) + pv(p_n, t_n)
            out_ref[0, pl.ds(c * CHUNK, CHUNK), :] = acc / denom
            lse_ref[0, pl.ds(c * CHUNK, CHUNK), :] = m + jnp.log(denom)


def _banded_attention(xys, C, Cr, padding):
    H, L, D = xys.shape
    Lp = L + padding
    nc = Lp // CHUNK
    G = ATTN_BLOCK
    gb = -(-nc // G)
    Lo = gb * G * CHUNK
    kern = functools.partial(_attn_kernel, nc=nc, C=C, Cr=Cr, G=G, L=L,
                             padding=padding)
    out, lse = pl.pallas_call(
        kern,
        grid=(H, gb),
        in_specs=[
            pl.BlockSpec((1, L, D), lambda h, g: (h, 0, 0)),
        ],
        out_specs=[
            pl.BlockSpec((1, G * CHUNK, Cr), lambda h, g: (h, g, 0)),
            pl.BlockSpec((1, G * CHUNK, 1), lambda h, g: (h, g, 0)),
        ],
        out_shape=[
            jax.ShapeDtypeStruct((H, Lo, Cr), jnp.float32),
            jax.ShapeDtypeStruct((H, Lo, 1), jnp.float32),
        ],
        compiler_params=pltpu.CompilerParams(
            dimension_semantics=("parallel", "parallel")),
    )(xys)
    return out[:, :Lp], lse[:, :Lp, 0]


def kernel(input, w_match, b_match, w_assembly, b_assembly):
    x = input
    N, _, Hh, Ww = x.shape
    L = Hh * Ww
    C = w_match.shape[0]
    Cr = w_assembly.shape[0]
    H = N_HASHES

    x_embed = _conv(x, w_match, b_match).reshape(N, C, L).transpose(0, 2, 1)
    y_embed = _conv(x, w_assembly, b_assembly).reshape(N, Cr, L).transpose(0, 2, 1)

    hash_buckets = min(L // CHUNK + (L // CHUNK) % 2, 128)
    rot = jax.random.normal(jax.random.key(42), (1, C, H, hash_buckets // 2),
                            dtype=x_embed.dtype)
    rot = jnp.broadcast_to(rot, (N, C, H, hash_buckets // 2))
    rotated = jnp.einsum('btf,bfhi->bhti', x_embed, rot)
    rotated = jnp.concatenate([rotated, -rotated], axis=-1)
    codes = jnp.argmax(rotated, axis=-1).astype(jnp.int32)

    T = RANK_TILE
    rank_pad = (-L) % T
    codes_lh = codes.reshape(H, L).transpose(1, 0)
    if rank_pad:
        codes_lh = jnp.concatenate(
            [codes_lh, jnp.full((rank_pad, H), 128, jnp.int32)], axis=0)
    nt = codes_lh.shape[0] // T
    pos4 = _counting_rank(codes_lh.reshape(nt, T, H), L, H)
    pos = pos4.reshape(nt * T, H)[:L].transpose(1, 0)
    undo_sort = pos.reshape(N, H * L)

    padding = CHUNK - L % CHUNK if L % CHUNK != 0 else 0
    Lp = L + padding

    spos = pos.reshape(1, H * L)

    d_pad = (-(C + Cr)) % 128
    xy_embed = jnp.concatenate(
        [x_embed, y_embed]
        + ([jnp.zeros((N, L, d_pad), x_embed.dtype)] if d_pad else []),
        axis=-1)
    D = C + Cr + d_pad
    xys_flat = _sc_sort_scatter(xy_embed.reshape(L, D), spos, H)
    xys = xys_flat.reshape(H, L, D)

    ret, bucket_score = _banded_attention(xys, C, Cr, padding)

    if padding:
        ret = ret[:, :-padding, :]
        bucket_score = bucket_score[:, :-padding]
    ret = ret.reshape(N, H * L, Cr)
    bucket_score = bucket_score.reshape(N, H * L)

    ret = jnp.take_along_axis(ret, undo_sort[:, :, None], axis=1)
    bucket_score = jnp.take_along_axis(bucket_score, undo_sort, axis=1)

    ret = ret.reshape(N, H, L, Cr)
    bucket_score = bucket_score.reshape(N, H, L, 1)
    probs = jax.nn.softmax(bucket_score, axis=1)
    ret = jnp.sum(ret * probs, axis=1)
    ret = ret.transpose(0, 2, 1).reshape(N, Cr, Hh, Ww) * RES_SCALE + x
    return ret


# speedup vs baseline: 2.0906x; 1.0355x over previous
import functools

import jax
import jax.numpy as jnp
from jax.experimental import pallas as pl
from jax.experimental.pallas import tpu as pltpu
from jax.experimental.pallas import tpu_sc as plsc

N_HASHES = 4
CHUNK = 144
RES_SCALE = 1.0
REDUCTION = 4
RANK_TILE = 1024
ATTN_BLOCK = 8


def _conv(x, w, b):
    out = jax.lax.conv_general_dilated(x, w, (1, 1), 'SAME',
                                       dimension_numbers=('NCHW', 'OIHW', 'NCHW'))
    return out + b[None, :, None, None]


def _rank_kernel(codes_ref, tri_ref, r1_ref, hist_ref, carry_ref, *, nb, H):
    t = pl.program_id(0)

    @pl.when(t == 0)
    def _():
        carry_ref[...] = jnp.zeros_like(carry_ref)

    c = codes_ref[0, :, :]
    lanes = jax.lax.broadcasted_iota(jnp.int32, (1, nb), 1)
    oh = jnp.concatenate(
        [(c[:, h:h + 1] == lanes) for h in range(H)],
        axis=1).astype(jnp.float32)

    cum = jax.lax.dot_general(tri_ref[...], oh.astype(jnp.bfloat16),
                              (((1,), (0,)), ((), ())),
                              preferred_element_type=jnp.float32)
    carry = carry_ref[0:1, :]
    tot = oh * (cum + carry)
    r1 = jnp.concatenate(
        [jnp.sum(tot[:, h * nb:(h + 1) * nb], axis=1, keepdims=True)
         for h in range(H)], axis=1)
    r1_ref[0, :, :] = r1.astype(jnp.int32)

    new_carry = carry + jnp.sum(oh, axis=0, keepdims=True)
    carry_ref[0:1, :] = new_carry
    hist_ref[0, :, :] = new_carry


def _pos_kernel(codes_ref, r1_ref, base_ref, pos_ref, *, nb, H):
    c = codes_ref[0, :, :]
    lanes = jax.lax.broadcasted_iota(jnp.int32, (1, nb), 1)
    oh = jnp.concatenate(
        [(c[:, h:h + 1] == lanes) for h in range(H)],
        axis=1).astype(jnp.float32)
    base = base_ref[0, :, :].astype(jnp.float32)
    bl = oh * base
    b = jnp.concatenate(
        [jnp.sum(bl[:, h * nb:(h + 1) * nb], axis=1, keepdims=True)
         for h in range(H)], axis=1)
    pos_ref[0, :, :] = b.astype(jnp.int32) + r1_ref[0, :, :]


def _counting_rank(codes4, L, H):
    nt, T, _ = codes4.shape
    nb = 128
    ii = jax.lax.broadcasted_iota(jnp.int32, (T, T), 0)
    jj = jax.lax.broadcasted_iota(jnp.int32, (T, T), 1)
    tri = (jj < ii).astype(jnp.bfloat16)

    r1, hist = pl.pallas_call(
        functools.partial(_rank_kernel, nb=nb, H=H),
        grid=(nt,),
        in_specs=[
            pl.BlockSpec((1, T, H), lambda t: (t, 0, 0)),
            pl.BlockSpec((T, T), lambda t: (0, 0)),
        ],
        out_specs=[
            pl.BlockSpec((1, T, H), lambda t: (t, 0, 0)),
            pl.BlockSpec((1, 1, H * nb), lambda t: (0, 0, 0)),
        ],
        out_shape=[
            jax.ShapeDtypeStruct((nt, T, H), jnp.int32),
            jax.ShapeDtypeStruct((1, 1, H * nb), jnp.float32),
        ],
        scratch_shapes=[pltpu.VMEM((8, H * nb), jnp.float32)],
    )(codes4, tri)

    totals = hist.reshape(H, nb)
    bucket_base = (jnp.cumsum(totals, axis=1) - totals
                   + (jnp.arange(H, dtype=jnp.float32) * L)[:, None])
    base3 = bucket_base.astype(jnp.int32).reshape(1, 1, H * nb)

    pos = pl.pallas_call(
        functools.partial(_pos_kernel, nb=nb, H=H),
        grid=(nt,),
        in_specs=[
            pl.BlockSpec((1, T, H), lambda t: (t, 0, 0)),
            pl.BlockSpec((1, T, H), lambda t: (t, 0, 0)),
            pl.BlockSpec((1, 1, H * nb), lambda t: (0, 0, 0)),
        ],
        out_specs=pl.BlockSpec((1, T, H), lambda t: (t, 0, 0)),
        out_shape=jax.ShapeDtypeStruct((nt, T, H), jnp.int32),
    )(codes4, r1, base3)
    return pos


SC_WINDOW = 128


def _sc_sort_scatter(rows, spos, H):
    L, D = rows.shape
    n_idx = H * L
    nxb = L // SC_WINDOW

    @functools.partial(
        pl.kernel,
        out_type=jax.ShapeDtypeStruct((H * L, D), rows.dtype),
        mesh=plsc.VectorSubcoreMesh(core_axis_name="core",
                                    subcore_axis_name="subcore"),
        scratch_types=[])
    def scatter_kernel(x_hbm, i_hbm, o_hbm):
        def body(x_vmem, i_vmem):
            pltpu.sync_copy(x_vmem, o_hbm.at[i_vmem.at[0]])

        pltpu.emit_pipeline(
            body,
            grid=(n_idx // SC_WINDOW,),
            in_specs=[
                pl.BlockSpec((SC_WINDOW, D),
                             index_map=lambda i: (jax.lax.rem(i, nxb), 0)),
                pl.BlockSpec((1, SC_WINDOW), index_map=lambda i: (0, i)),
            ],
            out_specs=[],
            core_axis_name=('core', 'subcore'),
            dimension_semantics=(pltpu.PARALLEL,),
        )(x_hbm, i_hbm)

    return scatter_kernel(rows, spos)


def _attn_kernel(xys_ref, out_ref, lse_ref, *, nc, C, Cr, G, L, padding):
    g = pl.program_id(1)

    def load(j):
        if padding:
            s1 = CHUNK - padding
            a = jnp.where(j == nc - 1, L - s1, j * CHUNK)
            b = jnp.where(j == nc - 1, L - padding, j * CHUNK + s1)
            return jnp.concatenate(
                [xys_ref[0, pl.ds(a, s1), :],
                 xys_ref[0, pl.ds(b, padding), :]], axis=0)
        return xys_ref[0, pl.ds(j * CHUNK, CHUNK), :]

    for c in range(G):
        k = g * G + c

        @pl.when(k < nc)
        def _(k=k, c=c):
            prev = jax.lax.rem(k - 1 + nc, nc)
            nxt = jax.lax.rem(k + 1, nc)
            t_c = load(k)
            kwin = jnp.concatenate([t_c, load(prev), load(nxt)],
                                   axis=0)
            q = t_c[:, :C]
            kn = kwin[:, :C] * kwin[:, C + Cr:C + Cr + 1]
            s = jax.lax.dot_general(q, kn, (((1,), (1,)), ((), ())),
                                    preferred_element_type=jnp.float32)
            m = jnp.max(s, axis=1)[:, None]
            p = jnp.exp(s - m)
            denom = jnp.sum(p, axis=1)[:, None]
            acc = jax.lax.dot_general(p, kwin[:, C:C + Cr],
                                      (((1,), (0,)), ((), ())),
                                      preferred_element_type=jnp.float32)
            out_ref[0, pl.ds(c * CHUNK, CHUNK), :] = acc / denom
            lse_ref[0, pl.ds(c * CHUNK, CHUNK), :] = m + jnp.log(denom)


def _banded_attention(xys, C, Cr, padding):
    H, L, D = xys.shape
    Lp = L + padding
    nc = Lp // CHUNK
    G = ATTN_BLOCK
    gb = -(-nc // G)
    Lo = gb * G * CHUNK
    kern = functools.partial(_attn_kernel, nc=nc, C=C, Cr=Cr, G=G, L=L,
                             padding=padding)
    out, lse = pl.pallas_call(
        kern,
        grid=(H, gb),
        in_specs=[
            pl.BlockSpec((1, L, D), lambda h, g: (h, 0, 0)),
        ],
        out_specs=[
            pl.BlockSpec((1, G * CHUNK, Cr), lambda h, g: (h, g, 0)),
            pl.BlockSpec((1, G * CHUNK, 1), lambda h, g: (h, g, 0)),
        ],
        out_shape=[
            jax.ShapeDtypeStruct((H, Lo, Cr), jnp.float32),
            jax.ShapeDtypeStruct((H, Lo, 1), jnp.float32),
        ],
        compiler_params=pltpu.CompilerParams(
            dimension_semantics=("parallel", "parallel")),
    )(xys)
    return out[:, :Lp], lse[:, :Lp, 0]


def kernel(input, w_match, b_match, w_assembly, b_assembly):
    x = input
    N, _, Hh, Ww = x.shape
    L = Hh * Ww
    C = w_match.shape[0]
    Cr = w_assembly.shape[0]
    H = N_HASHES

    x_embed = _conv(x, w_match, b_match).reshape(N, C, L).transpose(0, 2, 1)
    y_embed = _conv(x, w_assembly, b_assembly).reshape(N, Cr, L).transpose(0, 2, 1)

    hash_buckets = min(L // CHUNK + (L // CHUNK) % 2, 128)
    rot = jax.random.normal(jax.random.key(42), (1, C, H, hash_buckets // 2),
                            dtype=x_embed.dtype)
    rot = jnp.broadcast_to(rot, (N, C, H, hash_buckets // 2))
    rotated = jnp.einsum('btf,bfhi->bhti', x_embed, rot)
    rotated = jnp.concatenate([rotated, -rotated], axis=-1)
    codes = jnp.argmax(rotated, axis=-1).astype(jnp.int32)

    T = RANK_TILE
    rank_pad = (-L) % T
    codes_lh = codes.reshape(H, L).transpose(1, 0)
    if rank_pad:
        codes_lh = jnp.concatenate(
            [codes_lh, jnp.full((rank_pad, H), 128, jnp.int32)], axis=0)
    nt = codes_lh.shape[0] // T
    pos4 = _counting_rank(codes_lh.reshape(nt, T, H), L, H)
    pos = pos4.reshape(nt * T, H)[:L].transpose(1, 0)
    undo_sort = pos.reshape(N, H * L)

    padding = CHUNK - L % CHUNK if L % CHUNK != 0 else 0
    Lp = L + padding

    spos = pos.reshape(1, H * L)

    rnorm = 1.0 / jnp.maximum(
        jnp.sqrt(jnp.sum(x_embed * x_embed, axis=-1, keepdims=True)), 5e-05)
    d_pad = (-(C + Cr + 1)) % 128
    xy_embed = jnp.concatenate(
        [x_embed, y_embed, rnorm]
        + ([jnp.zeros((N, L, d_pad), x_embed.dtype)] if d_pad else []),
        axis=-1)
    D = C + Cr + 1 + d_pad
    xys_flat = _sc_sort_scatter(xy_embed.reshape(L, D), spos, H)
    xys = xys_flat.reshape(H, L, D)

    ret, bucket_score = _banded_attention(xys, C, Cr, padding)

    if padding:
        ret = ret[:, :-padding, :]
        bucket_score = bucket_score[:, :-padding]
    ret = ret.reshape(N, H * L, Cr)
    bucket_score = bucket_score.reshape(N, H * L)

    ret = jnp.take_along_axis(ret, undo_sort[:, :, None], axis=1)
    bucket_score = jnp.take_along_axis(bucket_score, undo_sort, axis=1)

    ret = ret.reshape(N, H, L, Cr)
    bucket_score = bucket_score.reshape(N, H, L, 1)
    probs = jax.nn.softmax(bucket_score, axis=1)
    ret = jnp.sum(ret * probs, axis=1)
    ret = ret.transpose(0, 2, 1).reshape(N, Cr, Hh, Ww) * RES_SCALE + x
    return ret


# speedup vs baseline: 2.1407x; 1.0240x over previous
import functools

import jax
import jax.numpy as jnp
from jax.experimental import pallas as pl
from jax.experimental.pallas import tpu as pltpu
from jax.experimental.pallas import tpu_sc as plsc

N_HASHES = 4
CHUNK = 144
RES_SCALE = 1.0
REDUCTION = 4
RANK_TILE = 1024
ATTN_BLOCK = 8


def _conv(x, w, b):
    out = jax.lax.conv_general_dilated(x, w, (1, 1), 'SAME',
                                       dimension_numbers=('NCHW', 'OIHW', 'NCHW'))
    return out + b[None, :, None, None]


def _rank_kernel(codes_ref, tri_ref, r1_ref, hist_ref, carry_ref, *, nb, H):
    t = pl.program_id(0)

    @pl.when(t == 0)
    def _():
        carry_ref[...] = jnp.zeros_like(carry_ref)

    c = codes_ref[0, :, :]
    lanes = jax.lax.broadcasted_iota(jnp.int32, (1, nb), 1)
    oh = jnp.concatenate(
        [(c[:, h:h + 1] == lanes) for h in range(H)],
        axis=1).astype(jnp.float32)

    cum = jax.lax.dot_general(tri_ref[...], oh.astype(jnp.bfloat16),
                              (((1,), (0,)), ((), ())),
                              preferred_element_type=jnp.float32)
    carry = carry_ref[0:1, :]
    tot = oh * (cum + carry)
    r1 = jnp.concatenate(
        [jnp.sum(tot[:, h * nb:(h + 1) * nb], axis=1, keepdims=True)
         for h in range(H)], axis=1)
    r1_ref[0, :, :] = r1.astype(jnp.int32)

    new_carry = carry + jnp.sum(oh, axis=0, keepdims=True)
    carry_ref[0:1, :] = new_carry
    hist_ref[0, :, :] = new_carry


def _pos_kernel(codes_ref, r1_ref, base_ref, pos_ref, *, nb, H):
    c = codes_ref[0, :, :]
    lanes = jax.lax.broadcasted_iota(jnp.int32, (1, nb), 1)
    oh = jnp.concatenate(
        [(c[:, h:h + 1] == lanes) for h in range(H)],
        axis=1).astype(jnp.float32)
    base = base_ref[0, :, :].astype(jnp.float32)
    bl = oh * base
    b = jnp.concatenate(
        [jnp.sum(bl[:, h * nb:(h + 1) * nb], axis=1, keepdims=True)
         for h in range(H)], axis=1)
    pos_ref[0, :, :] = b.astype(jnp.int32) + r1_ref[0, :, :]


def _counting_rank(codes4, L, H):
    nt, T, _ = codes4.shape
    nb = 128
    ii = jax.lax.broadcasted_iota(jnp.int32, (T, T), 0)
    jj = jax.lax.broadcasted_iota(jnp.int32, (T, T), 1)
    tri = (jj < ii).astype(jnp.bfloat16)

    r1, hist = pl.pallas_call(
        functools.partial(_rank_kernel, nb=nb, H=H),
        grid=(nt,),
        in_specs=[
            pl.BlockSpec((1, T, H), lambda t: (t, 0, 0)),
            pl.BlockSpec((T, T), lambda t: (0, 0)),
        ],
        out_specs=[
            pl.BlockSpec((1, T, H), lambda t: (t, 0, 0)),
            pl.BlockSpec((1, 1, H * nb), lambda t: (0, 0, 0)),
        ],
        out_shape=[
            jax.ShapeDtypeStruct((nt, T, H), jnp.int32),
            jax.ShapeDtypeStruct((1, 1, H * nb), jnp.float32),
        ],
        scratch_shapes=[pltpu.VMEM((8, H * nb), jnp.float32)],
    )(codes4, tri)

    totals = hist.reshape(H, nb)
    bucket_base = (jnp.cumsum(totals, axis=1) - totals
                   + (jnp.arange(H, dtype=jnp.float32) * L)[:, None])
    base3 = bucket_base.astype(jnp.int32).reshape(1, 1, H * nb)

    pos = pl.pallas_call(
        functools.partial(_pos_kernel, nb=nb, H=H),
        grid=(nt,),
        in_specs=[
            pl.BlockSpec((1, T, H), lambda t: (t, 0, 0)),
            pl.BlockSpec((1, T, H), lambda t: (t, 0, 0)),
            pl.BlockSpec((1, 1, H * nb), lambda t: (0, 0, 0)),
        ],
        out_specs=pl.BlockSpec((1, T, H), lambda t: (t, 0, 0)),
        out_shape=jax.ShapeDtypeStruct((nt, T, H), jnp.int32),
    )(codes4, r1, base3)
    return pos


SC_WINDOW = 128


def _sc_sort_scatter(rows, spos, H):
    L, D = rows.shape
    n_idx = H * L
    nxb = L // SC_WINDOW

    @functools.partial(
        pl.kernel,
        out_type=jax.ShapeDtypeStruct((H * L, D), rows.dtype),
        mesh=plsc.VectorSubcoreMesh(core_axis_name="core",
                                    subcore_axis_name="subcore"),
        scratch_types=[])
    def scatter_kernel(x_hbm, i_hbm, o_hbm):
        def body(x_vmem, i_vmem):
            pltpu.sync_copy(x_vmem, o_hbm.at[i_vmem.at[0]])

        pltpu.emit_pipeline(
            body,
            grid=(n_idx // SC_WINDOW,),
            in_specs=[
                pl.BlockSpec((SC_WINDOW, D),
                             index_map=lambda i: (jax.lax.rem(i, nxb), 0)),
                pl.BlockSpec((1, SC_WINDOW), index_map=lambda i: (0, i)),
            ],
            out_specs=[],
            core_axis_name=('core', 'subcore'),
            dimension_semantics=(pltpu.PARALLEL,),
        )(x_hbm, i_hbm)

    return scatter_kernel(rows, spos)


def _attn_kernel(xys_ref, out_ref, lse_ref, *, nc, C, Cr, G, L, padding):
    g = pl.program_id(1)

    def load(j):
        if padding:
            s1 = CHUNK - padding
            a = jnp.where(j == nc - 1, L - s1, j * CHUNK)
            b = jnp.where(j == nc - 1, L - padding, j * CHUNK + s1)
            return jnp.concatenate(
                [xys_ref[0, pl.ds(a, s1), :],
                 xys_ref[0, pl.ds(b, padding), :]], axis=0)
        return xys_ref[0, pl.ds(j * CHUNK, CHUNK), :]

    for c in range(G):
        k = g * G + c

        @pl.when(k < nc)
        def _(k=k, c=c):
            prev = jax.lax.rem(k - 1 + nc, nc)
            nxt = jax.lax.rem(k + 1, nc)
            t_c = load(k)
            q = t_c[:, :C]
            m = t_c[:, C + Cr + 1:C + Cr + 2]

            def block(t):
                kn = t[:, :C] * t[:, C + Cr:C + Cr + 1]
                s = jax.lax.dot_general(q, kn, (((1,), (1,)), ((), ())),
                                        preferred_element_type=jnp.float32)
                p = jnp.exp(s - m)
                a = jax.lax.dot_general(p, t[:, C:C + Cr],
                                        (((1,), (0,)), ((), ())),
                                        preferred_element_type=jnp.float32)
                return a, jnp.sum(p, axis=1, keepdims=True)

            a1, d1 = block(t_c)
            a2, d2 = block(load(prev))
            a3, d3 = block(load(nxt))
            acc = a1 + a2 + a3
            denom = d1 + d2 + d3
            out_ref[0, pl.ds(c * CHUNK, CHUNK), :] = acc / denom
            lse_ref[0, pl.ds(c * CHUNK, CHUNK), :] = m + jnp.log(denom)


def _banded_attention(xys, C, Cr, padding):
    H, L, D = xys.shape
    Lp = L + padding
    nc = Lp // CHUNK
    G = ATTN_BLOCK
    gb = -(-nc // G)
    Lo = gb * G * CHUNK
    kern = functools.partial(_attn_kernel, nc=nc, C=C, Cr=Cr, G=G, L=L,
                             padding=padding)
    out, lse = pl.pallas_call(
        kern,
        grid=(H, gb),
        in_specs=[
            pl.BlockSpec((1, L, D), lambda h, g: (h, 0, 0)),
        ],
        out_specs=[
            pl.BlockSpec((1, G * CHUNK, Cr), lambda h, g: (h, g, 0)),
            pl.BlockSpec((1, G * CHUNK, 1), lambda h, g: (h, g, 0)),
        ],
        out_shape=[
            jax.ShapeDtypeStruct((H, Lo, Cr), jnp.float32),
            jax.ShapeDtypeStruct((H, Lo, 1), jnp.float32),
        ],
        compiler_params=pltpu.CompilerParams(
            dimension_semantics=("parallel", "parallel")),
    )(xys)
    return out[:, :Lp], lse[:, :Lp, 0]


def kernel(input, w_match, b_match, w_assembly, b_assembly):
    x = input
    N, _, Hh, Ww = x.shape
    L = Hh * Ww
    C = w_match.shape[0]
    Cr = w_assembly.shape[0]
    H = N_HASHES

    x_embed = _conv(x, w_match, b_match).reshape(N, C, L).transpose(0, 2, 1)
    y_embed = _conv(x, w_assembly, b_assembly).reshape(N, Cr, L).transpose(0, 2, 1)

    hash_buckets = min(L // CHUNK + (L // CHUNK) % 2, 128)
    rot = jax.random.normal(jax.random.key(42), (1, C, H, hash_buckets // 2),
                            dtype=x_embed.dtype)
    rot = jnp.broadcast_to(rot, (N, C, H, hash_buckets // 2))
    rotated = jnp.einsum('btf,bfhi->bhti', x_embed, rot)
    rotated = jnp.concatenate([rotated, -rotated], axis=-1)
    codes = jnp.argmax(rotated, axis=-1).astype(jnp.int32)

    T = RANK_TILE
    rank_pad = (-L) % T
    codes_lh = codes.reshape(H, L).transpose(1, 0)
    if rank_pad:
        codes_lh = jnp.concatenate(
            [codes_lh, jnp.full((rank_pad, H), 128, jnp.int32)], axis=0)
    nt = codes_lh.shape[0] // T
    pos4 = _counting_rank(codes_lh.reshape(nt, T, H), L, H)
    pos = pos4.reshape(nt * T, H)[:L].transpose(1, 0)
    undo_sort = pos.reshape(N, H * L)

    padding = CHUNK - L % CHUNK if L % CHUNK != 0 else 0
    Lp = L + padding

    spos = pos.reshape(1, H * L)

    nrm_c = jnp.maximum(
        jnp.sqrt(jnp.sum(x_embed * x_embed, axis=-1, keepdims=True)), 5e-05)
    rnorm = 1.0 / nrm_c
    d_pad = (-(C + Cr + 2)) % 128
    xy_embed = jnp.concatenate(
        [x_embed, y_embed, rnorm, nrm_c]
        + ([jnp.zeros((N, L, d_pad), x_embed.dtype)] if d_pad else []),
        axis=-1)
    D = C + Cr + 2 + d_pad
    xys_flat = _sc_sort_scatter(xy_embed.reshape(L, D), spos, H)
    xys = xys_flat.reshape(H, L, D)

    ret, bucket_score = _banded_attention(xys, C, Cr, padding)

    if padding:
        ret = ret[:, :-padding, :]
        bucket_score = bucket_score[:, :-padding]
    ret = ret.reshape(N, H * L, Cr)
    bucket_score = bucket_score.reshape(N, H * L)

    ret = jnp.take_along_axis(ret, undo_sort[:, :, None], axis=1)
    bucket_score = jnp.take_along_axis(bucket_score, undo_sort, axis=1)

    ret = ret.reshape(N, H, L, Cr)
    bucket_score = bucket_score.reshape(N, H, L, 1)
    probs = jax.nn.softmax(bucket_score, axis=1)
    ret = jnp.sum(ret * probs, axis=1)
    ret = ret.transpose(0, 2, 1).reshape(N, Cr, Hh, Ww) * RES_SCALE + x
    return ret


# speedup vs baseline: 2.1540x; 1.0062x over previous
import functools

import jax
import jax.numpy as jnp
from jax.experimental import pallas as pl
from jax.experimental.pallas import tpu as pltpu
from jax.experimental.pallas import tpu_sc as plsc

N_HASHES = 4
CHUNK = 144
RES_SCALE = 1.0
REDUCTION = 4
RANK_TILE = 1024
ATTN_BLOCK = 16


def _conv(x, w, b):
    out = jax.lax.conv_general_dilated(x, w, (1, 1), 'SAME',
                                       dimension_numbers=('NCHW', 'OIHW', 'NCHW'))
    return out + b[None, :, None, None]


def _rank_kernel(codes_ref, tri_ref, r1_ref, hist_ref, carry_ref, *, nb, H):
    t = pl.program_id(0)

    @pl.when(t == 0)
    def _():
        carry_ref[...] = jnp.zeros_like(carry_ref)

    c = codes_ref[0, :, :]
    lanes = jax.lax.broadcasted_iota(jnp.int32, (1, nb), 1)
    oh = jnp.concatenate(
        [(c[:, h:h + 1] == lanes) for h in range(H)],
        axis=1).astype(jnp.float32)

    cum = jax.lax.dot_general(tri_ref[...], oh.astype(jnp.bfloat16),
                              (((1,), (0,)), ((), ())),
                              preferred_element_type=jnp.float32)
    carry = carry_ref[0:1, :]
    tot = oh * (cum + carry)
    r1 = jnp.concatenate(
        [jnp.sum(tot[:, h * nb:(h + 1) * nb], axis=1, keepdims=True)
         for h in range(H)], axis=1)
    r1_ref[0, :, :] = r1.astype(jnp.int32)

    new_carry = carry + jnp.sum(oh, axis=0, keepdims=True)
    carry_ref[0:1, :] = new_carry
    hist_ref[0, :, :] = new_carry


def _pos_kernel(codes_ref, r1_ref, base_ref, pos_ref, *, nb, H):
    c = codes_ref[0, :, :]
    lanes = jax.lax.broadcasted_iota(jnp.int32, (1, nb), 1)
    oh = jnp.concatenate(
        [(c[:, h:h + 1] == lanes) for h in range(H)],
        axis=1).astype(jnp.float32)
    base = base_ref[0, :, :].astype(jnp.float32)
    bl = oh * base
    b = jnp.concatenate(
        [jnp.sum(bl[:, h * nb:(h + 1) * nb], axis=1, keepdims=True)
         for h in range(H)], axis=1)
    pos_ref[0, :, :] = b.astype(jnp.int32) + r1_ref[0, :, :]


def _counting_rank(codes4, L, H):
    nt, T, _ = codes4.shape
    nb = 128
    ii = jax.lax.broadcasted_iota(jnp.int32, (T, T), 0)
    jj = jax.lax.broadcasted_iota(jnp.int32, (T, T), 1)
    tri = (jj < ii).astype(jnp.bfloat16)

    r1, hist = pl.pallas_call(
        functools.partial(_rank_kernel, nb=nb, H=H),
        grid=(nt,),
        in_specs=[
            pl.BlockSpec((1, T, H), lambda t: (t, 0, 0)),
            pl.BlockSpec((T, T), lambda t: (0, 0)),
        ],
        out_specs=[
            pl.BlockSpec((1, T, H), lambda t: (t, 0, 0)),
            pl.BlockSpec((1, 1, H * nb), lambda t: (0, 0, 0)),
        ],
        out_shape=[
            jax.ShapeDtypeStruct((nt, T, H), jnp.int32),
            jax.ShapeDtypeStruct((1, 1, H * nb), jnp.float32),
        ],
        scratch_shapes=[pltpu.VMEM((8, H * nb), jnp.float32)],
    )(codes4, tri)

    totals = hist.reshape(H, nb)
    bucket_base = (jnp.cumsum(totals, axis=1) - totals
                   + (jnp.arange(H, dtype=jnp.float32) * L)[:, None])
    base3 = bucket_base.astype(jnp.int32).reshape(1, 1, H * nb)

    pos = pl.pallas_call(
        functools.partial(_pos_kernel, nb=nb, H=H),
        grid=(nt,),
        in_specs=[
            pl.BlockSpec((1, T, H), lambda t: (t, 0, 0)),
            pl.BlockSpec((1, T, H), lambda t: (t, 0, 0)),
            pl.BlockSpec((1, 1, H * nb), lambda t: (0, 0, 0)),
        ],
        out_specs=pl.BlockSpec((1, T, H), lambda t: (t, 0, 0)),
        out_shape=jax.ShapeDtypeStruct((nt, T, H), jnp.int32),
    )(codes4, r1, base3)
    return pos


SC_WINDOW = 128


def _sc_sort_scatter(rows, spos, H):
    L, D = rows.shape
    n_idx = H * L
    nxb = L // SC_WINDOW

    @functools.partial(
        pl.kernel,
        out_type=jax.ShapeDtypeStruct((H * L, D), rows.dtype),
        mesh=plsc.VectorSubcoreMesh(core_axis_name="core",
                                    subcore_axis_name="subcore"),
        scratch_types=[])
    def scatter_kernel(x_hbm, i_hbm, o_hbm):
        def body(x_vmem, i_vmem):
            pltpu.sync_copy(x_vmem, o_hbm.at[i_vmem.at[0]])

        pltpu.emit_pipeline(
            body,
            grid=(n_idx // SC_WINDOW,),
            in_specs=[
                pl.BlockSpec((SC_WINDOW, D),
                             index_map=lambda i: (jax.lax.rem(i, nxb), 0)),
                pl.BlockSpec((1, SC_WINDOW), index_map=lambda i: (0, i)),
            ],
            out_specs=[],
            core_axis_name=('core', 'subcore'),
            dimension_semantics=(pltpu.PARALLEL,),
        )(x_hbm, i_hbm)

    return scatter_kernel(rows, spos)


def _attn_kernel(xys_ref, out_ref, lse_ref, *, nc, C, Cr, G, L, padding):
    g = pl.program_id(1)

    def load(j):
        if padding:
            s1 = CHUNK - padding
            a = jnp.where(j == nc - 1, L - s1, j * CHUNK)
            b = jnp.where(j == nc - 1, L - padding, j * CHUNK + s1)
            return jnp.concatenate(
                [xys_ref[0, pl.ds(a, s1), :],
                 xys_ref[0, pl.ds(b, padding), :]], axis=0)
        return xys_ref[0, pl.ds(j * CHUNK, CHUNK), :]

    for c in range(G):
        k = g * G + c

        @pl.when(k < nc)
        def _(k=k, c=c):
            prev = jax.lax.rem(k - 1 + nc, nc)
            nxt = jax.lax.rem(k + 1, nc)
            t_c = load(k)
            q = t_c[:, :C]
            m = t_c[:, C + Cr + 1:C + Cr + 2]

            def block(t):
                kn = t[:, :C] * t[:, C + Cr:C + Cr + 1]
                s = jax.lax.dot_general(q, kn, (((1,), (1,)), ((), ())),
                                        preferred_element_type=jnp.float32)
                p = jnp.exp(s - m)
                a = jax.lax.dot_general(p, t[:, C:C + Cr],
                                        (((1,), (0,)), ((), ())),
                                        preferred_element_type=jnp.float32)
                return a, jnp.sum(p, axis=1, keepdims=True)

            a1, d1 = block(t_c)
            a2, d2 = block(load(prev))
            a3, d3 = block(load(nxt))
            acc = a1 + a2 + a3
            denom = d1 + d2 + d3
            out_ref[0, pl.ds(c * CHUNK, CHUNK), :] = acc / denom
            lse_ref[0, pl.ds(c * CHUNK, CHUNK), :] = m + jnp.log(denom)


def _banded_attention(xys, C, Cr, padding):
    H, L, D = xys.shape
    Lp = L + padding
    nc = Lp // CHUNK
    G = ATTN_BLOCK
    gb = -(-nc // G)
    Lo = gb * G * CHUNK
    kern = functools.partial(_attn_kernel, nc=nc, C=C, Cr=Cr, G=G, L=L,
                             padding=padding)
    out, lse = pl.pallas_call(
        kern,
        grid=(H, gb),
        in_specs=[
            pl.BlockSpec((1, L, D), lambda h, g: (h, 0, 0)),
        ],
        out_specs=[
            pl.BlockSpec((1, G * CHUNK, Cr), lambda h, g: (h, g, 0)),
            pl.BlockSpec((1, G * CHUNK, 1), lambda h, g: (h, g, 0)),
        ],
        out_shape=[
            jax.ShapeDtypeStruct((H, Lo, Cr), jnp.float32),
            jax.ShapeDtypeStruct((H, Lo, 1), jnp.float32),
        ],
        compiler_params=pltpu.CompilerParams(
            dimension_semantics=("parallel", "parallel")),
    )(xys)
    return out[:, :Lp], lse[:, :Lp, 0]


def kernel(input, w_match, b_match, w_assembly, b_assembly):
    x = input
    N, _, Hh, Ww = x.shape
    L = Hh * Ww
    C = w_match.shape[0]
    Cr = w_assembly.shape[0]
    H = N_HASHES

    x_embed = _conv(x, w_match, b_match).reshape(N, C, L).transpose(0, 2, 1)
    y_embed = _conv(x, w_assembly, b_assembly).reshape(N, Cr, L).transpose(0, 2, 1)

    hash_buckets = min(L // CHUNK + (L // CHUNK) % 2, 128)
    rot = jax.random.normal(jax.random.key(42), (1, C, H, hash_buckets // 2),
                            dtype=x_embed.dtype)
    rot = jnp.broadcast_to(rot, (N, C, H, hash_buckets // 2))
    rotated = jnp.einsum('btf,bfhi->bhti', x_embed, rot)
    rotated = jnp.concatenate([rotated, -rotated], axis=-1)
    codes = jnp.argmax(rotated, axis=-1).astype(jnp.int32)

    T = RANK_TILE
    rank_pad = (-L) % T
    codes_lh = codes.reshape(H, L).transpose(1, 0)
    if rank_pad:
        codes_lh = jnp.concatenate(
            [codes_lh, jnp.full((rank_pad, H), 128, jnp.int32)], axis=0)
    nt = codes_lh.shape[0] // T
    pos4 = _counting_rank(codes_lh.reshape(nt, T, H), L, H)
    pos = pos4.reshape(nt * T, H)[:L].transpose(1, 0)
    undo_sort = pos.reshape(N, H * L)

    padding = CHUNK - L % CHUNK if L % CHUNK != 0 else 0
    Lp = L + padding

    spos = pos.reshape(1, H * L)

    nrm_c = jnp.maximum(
        jnp.sqrt(jnp.sum(x_embed * x_embed, axis=-1, keepdims=True)), 5e-05)
    rnorm = 1.0 / nrm_c
    d_pad = (-(C + Cr + 2)) % 128
    xy_embed = jnp.concatenate(
        [x_embed, y_embed, rnorm, nrm_c]
        + ([jnp.zeros((N, L, d_pad), x_embed.dtype)] if d_pad else []),
        axis=-1)
    D = C + Cr + 2 + d_pad
    xys_flat = _sc_sort_scatter(xy_embed.reshape(L, D), spos, H)
    xys = xys_flat.reshape(H, L, D)

    ret, bucket_score = _banded_attention(xys, C, Cr, padding)

    if padding:
        ret = ret[:, :-padding, :]
        bucket_score = bucket_score[:, :-padding]
    ret = ret.reshape(N, H * L, Cr)
    bucket_score = bucket_score.reshape(N, H * L)

    ret = jnp.take_along_axis(ret, undo_sort[:, :, None], axis=1)
    bucket_score = jnp.take_along_axis(bucket_score, undo_sort, axis=1)

    ret = ret.reshape(N, H, L, Cr)
    bucket_score = bucket_score.reshape(N, H, L, 1)
    probs = jax.nn.softmax(bucket_score, axis=1)
    ret = jnp.sum(ret * probs, axis=1)
    ret = ret.transpose(0, 2, 1).reshape(N, Cr, Hh, Ww) * RES_SCALE + x
    return ret
